# Initial kernel scaffold; baseline (speedup 1.0000x reference)
import functools

import jax
import jax.numpy as jnp
from jax import lax
from jax.experimental import pallas as pl
from jax.experimental.pallas import tpu as pltpu
from jax.experimental.pallas import tpu_sc as plsc

N_NODES = 10000
D = 128
DH = 64
NPAD = 10240
ROWS_PT = NPAD // 16
B = 128
BPT = 158
EPT = BPT * B
E_PAD = 16 * EPT
PAD_NODE = NPAD - 1


def _sc_body(x_cat, srcb, dstb, xs, t1, h1, h2,
             src_v, dst_v, gbuf, dis_v, deg_v, stage_v, tmp_v, tmp2_v,
             deg_sh, dis_sh, acc1_sh, acc2_sh, sem):
    c = lax.axis_index("c")
    s = lax.axis_index("s")
    base = s * ROWS_PT
    zeros16 = jnp.zeros((16,), jnp.float32)
    ones16 = jnp.ones((16,), jnp.float32)

    pltpu.sync_copy(srcb.at[pl.ds((c * 16 + s) * BPT, BPT)], src_v)
    pltpu.sync_copy(dstb.at[pl.ds(s * BPT, BPT)], dst_v)

    def _zero_deg(i, _):
        deg_v[pl.ds(i * 16, 16)] = zeros16
        return 0
    lax.fori_loop(0, NPAD // 16, _zero_deg, 0)

    def _count(j, _):
        def _count_in(k, _):
            idx = dst_v[j, pl.ds(k * 16, 16)]
            plsc.addupdate_scatter(deg_v, [idx], ones16)
            return 0
        lax.fori_loop(0, B // 16, _count_in, 0)
        return 0
    lax.fori_loop(0, BPT, _count, 0)

    pltpu.sync_copy(deg_v, deg_sh.at[s])
    plsc.subcore_barrier()

    def _zero_tmp(i, _):
        tmp_v[pl.ds(i * 16, 16)] = zeros16
        return 0
    lax.fori_loop(0, ROWS_PT // 16, _zero_tmp, 0)

    def _red(k, _):
        pltpu.sync_copy(deg_sh.at[k, pl.ds(base, ROWS_PT)], tmp2_v)
        def _acc(i, _):
            tmp_v[pl.ds(i * 16, 16)] = (tmp_v[pl.ds(i * 16, 16)]
                                        + tmp2_v[pl.ds(i * 16, 16)])
            return 0
        lax.fori_loop(0, ROWS_PT // 16, _acc, 0)
        return 0
    lax.fori_loop(0, 16, _red, 0)

    def _mkdis(i, _):
        d = tmp_v[pl.ds(i * 16, 16)]
        bi = plsc.bitcast(d, jnp.int32)
        y = plsc.bitcast(jnp.int32(0x5F3759DF) - (bi >> 1), jnp.float32)
        for _ in range(3):
            y = y * (1.5 - 0.5 * d * y * y)
        nid = base + i * 16 + lax.iota(jnp.int32, 16)
        y = jnp.where((nid < N_NODES) & (d > 0.0), y, 0.0)
        tmp2_v[pl.ds(i * 16, 16)] = y
        return 0
    lax.fori_loop(0, ROWS_PT // 16, _mkdis, 0)

    pltpu.sync_copy(tmp2_v, dis_sh.at[pl.ds(base, ROWS_PT)])
    plsc.subcore_barrier()
    pltpu.sync_copy(dis_sh, dis_v)

    def _zero_stage(r, _):
        for k in range(DH // 16):
            stage_v[r, pl.ds(k * 16, 16)] = zeros16
        return 0
    lax.fori_loop(0, ROWS_PT, _zero_stage, 0)
    pltpu.sync_copy(stage_v, acc1_sh.at[pl.ds(base, ROWS_PT)])
    pltpu.sync_copy(stage_v, acc2_sh.at[pl.ds(base, ROWS_PT)])

    @pl.when(s < 15)
    def _():
        pltpu.sync_copy(x_cat.at[pl.ds(c * N_NODES + base, ROWS_PT)], stage_v)

    @pl.when(s == 15)
    def _():
        pltpu.sync_copy(x_cat.at[pl.ds(c * N_NODES + 15 * ROWS_PT,
                                       N_NODES - 15 * ROWS_PT)],
                        stage_v.at[pl.ds(0, N_NODES - 15 * ROWS_PT)])

    def _scale(r, _):
        sc = dis_v[base + r]
        for k in range(DH // 16):
            stage_v[r, pl.ds(k * 16, 16)] = stage_v[r, pl.ds(k * 16, 16)] * sc
        return 0

    lax.fori_loop(0, ROWS_PT, _scale, 0)
    pltpu.sync_copy(stage_v, xs.at[pl.ds(c * NPAD + base, ROWS_PT)])
    plsc.subcore_barrier()

    def _run_layer(table, acc_sh):
        def _blk(j, _):
            pltpu.async_copy(table.at[src_v.at[j]], gbuf, sem).wait()
            pltpu.sync_copy(gbuf, acc_sh.at[dst_v.at[j]], add=True)
            return 0
        lax.fori_loop(0, BPT, _blk, 0)

    def _rescale_out(acc_sh, out1, out2):
        pltpu.sync_copy(acc_sh.at[pl.ds(base, ROWS_PT)], stage_v)
        lax.fori_loop(0, ROWS_PT, _scale, 0)
        pltpu.sync_copy(stage_v, out1.at[pl.ds(c * NPAD + base, ROWS_PT)])
        if out2 is not None:
            lax.fori_loop(0, ROWS_PT, _scale, 0)
            pltpu.sync_copy(stage_v, out2.at[pl.ds(c * NPAD + base, ROWS_PT)])

    _run_layer(xs, acc1_sh)
    plsc.subcore_barrier()
    _rescale_out(acc1_sh, h1, t1)
    plsc.subcore_barrier()
    _run_layer(t1, acc2_sh)
    plsc.subcore_barrier()
    _rescale_out(acc2_sh, h2, None)


def _make_sc_kernel():
    mesh = plsc.VectorSubcoreMesh(core_axis_name="c", subcore_axis_name="s",
                                  num_cores=2, num_subcores=16)
    hbm_out = jax.ShapeDtypeStruct((2 * NPAD, DH), jnp.float32)
    return pl.kernel(
        _sc_body,
        out_type=(hbm_out, hbm_out, hbm_out, hbm_out),
        mesh=mesh,
        scratch_types=[
            pltpu.VMEM((BPT, B), jnp.int32),
            pltpu.VMEM((BPT, B), jnp.int32),
            pltpu.VMEM((B, DH), jnp.float32),
            pltpu.VMEM((NPAD,), jnp.float32),
            pltpu.VMEM((NPAD,), jnp.float32),
            pltpu.VMEM((ROWS_PT, DH), jnp.float32),
            pltpu.VMEM((ROWS_PT,), jnp.float32),
            pltpu.VMEM((ROWS_PT,), jnp.float32),
            pltpu.VMEM_SHARED((16, NPAD), jnp.float32),
            pltpu.VMEM_SHARED((NPAD,), jnp.float32),
            pltpu.VMEM_SHARED((NPAD, DH), jnp.float32),
            pltpu.VMEM_SHARED((NPAD, DH), jnp.float32),
            pltpu.SemaphoreType.DMA,
        ],
    )


def _combine_body(alpha_ref, x_ref, h1l_ref, h1h_ref, h2l_ref, h2h_ref, o_ref):
    a0 = alpha_ref[0]
    a1 = alpha_ref[1]
    a2 = alpha_ref[2]
    o_ref[:, :DH] = a0 * x_ref[:, :DH] + a1 * h1l_ref[...] + a2 * h2l_ref[...]
    o_ref[:, DH:] = a0 * x_ref[:, DH:] + a1 * h1h_ref[...] + a2 * h2h_ref[...]


def _combine(alpha, x, h1, h2):
    R = 2000
    grid = (N_NODES // R,)
    half = lambda i: (i, 0)
    return pl.pallas_call(
        _combine_body,
        grid=grid,
        in_specs=[
            pl.BlockSpec(memory_space=pltpu.SMEM),
            pl.BlockSpec((R, D), half),
            pl.BlockSpec((R, DH), half),
            pl.BlockSpec((R, DH), half),
            pl.BlockSpec((R, DH), half),
            pl.BlockSpec((R, DH), half),
        ],
        out_specs=pl.BlockSpec((R, D), half),
        out_shape=jax.ShapeDtypeStruct((N_NODES, D), jnp.float32),
    )(alpha, x, h1[:N_NODES], h1[NPAD:NPAD + N_NODES],
      h2[:N_NODES], h2[NPAD:NPAD + N_NODES])


def kernel(x, edge_index, alpha):
    ei = edge_index.astype(jnp.int32)
    pad = jnp.full((E_PAD - ei.shape[1],), PAD_NODE, jnp.int32)
    src = jnp.concatenate([ei[0], pad])
    dst = jnp.concatenate([ei[1], pad]).reshape(16 * BPT, B)
    srcb = jnp.concatenate(
        [src.reshape(16 * BPT, B), src.reshape(16 * BPT, B) + NPAD])
    x_cat = jnp.concatenate([x[:, :DH], x[:, DH:]], axis=0)
    xs, t1, h1, h2 = _make_sc_kernel()(x_cat, srcb, dst)
    del xs, t1
    return _combine(alpha, x, h1, h2)


# speedup vs baseline: 6.2142x; 6.2142x over previous
import jax
import jax.numpy as jnp
from jax import lax
from jax.experimental import pallas as pl
from jax.experimental.pallas import tpu as pltpu
from jax.experimental.pallas import tpu_sc as plsc

N_NODES = 10000
D = 128
DQ = 32
NPAD = 10240
ROWS_PT = NPAD // 16
B = 128
BPT = 160
E_PAD = 16 * BPT * B
PAD_NODE = NPAD - 1


def _sc_body(x_cat, edges, xs, t1, h1, h2, degp,
             pk_v, src_v, dst_v, gbuf, dis_v, deg_v, stage_v, tmp_v,
             acc_sh, sem):
    c = lax.axis_index("c")
    s = lax.axis_index("s")
    base = s * ROWS_PT
    zeros16 = jnp.zeros((16,), jnp.float32)
    ones16 = jnp.ones((16,), jnp.float32)

    pltpu.sync_copy(edges.at[pl.ds(s * BPT, BPT)], pk_v)

    def _mk_dst(j, _):
        def _in(k, _):
            v = pk_v[j, pl.ds(k * 16, 16)]
            dst_v[j, pl.ds(k * 16, 16)] = v >> 14
            return 0
        lax.fori_loop(0, B // 16, _in, 0)
        return 0
    lax.fori_loop(0, BPT, _mk_dst, 0)

    def _mk_src(off):
        def _o(j, _):
            def _in(k, _):
                v = pk_v[j, pl.ds(k * 16, 16)]
                src_v[j, pl.ds(k * 16, 16)] = (v & 0x3FFF) + off
                return 0
            lax.fori_loop(0, B // 16, _in, 0)
            return 0
        lax.fori_loop(0, BPT, _o, 0)

    def _zero_deg(i, _):
        deg_v[pl.ds(i * 16, 16)] = zeros16
        return 0
    lax.fori_loop(0, NPAD // 16, _zero_deg, 0)

    def _count(j, _):
        def _in(k, _):
            idx = dst_v[j, pl.ds(k * 16, 16)]
            plsc.addupdate_scatter(deg_v, [idx], ones16)
            return 0
        lax.fori_loop(0, B // 16, _in, 0)
        return 0
    lax.fori_loop(0, BPT, _count, 0)

    pltpu.sync_copy(deg_v, degp.at[c * 16 + s])
    plsc.subcore_barrier()

    def _zero_dis(i, _):
        dis_v[pl.ds(i * 16, 16)] = zeros16
        return 0
    lax.fori_loop(0, ROWS_PT // 16, _zero_dis, 0)

    def _red(k, _):
        pltpu.sync_copy(degp.at[c * 16 + k, pl.ds(base, ROWS_PT)], tmp_v)
        def _acc(i, _):
            dis_v[pl.ds(i * 16, 16)] = (dis_v[pl.ds(i * 16, 16)]
                                        + tmp_v[pl.ds(i * 16, 16)])
            return 0
        lax.fori_loop(0, ROWS_PT // 16, _acc, 0)
        return 0
    lax.fori_loop(0, 16, _red, 0)

    def _mkdis(i, _):
        d = dis_v[pl.ds(i * 16, 16)]
        bi = plsc.bitcast(d, jnp.int32)
        y = plsc.bitcast(jnp.int32(0x5F3759DF) - (bi >> 1), jnp.float32)
        for _ in range(3):
            y = y * (1.5 - 0.5 * d * y * y)
        nid = base + i * 16 + lax.iota(jnp.int32, 16)
        y = jnp.where((nid < N_NODES) & (d > 0.0), y, 0.0)
        dis_v[pl.ds(i * 16, 16)] = y
        return 0
    lax.fori_loop(0, ROWS_PT // 16, _mkdis, 0)

    def _zero_stage(r, _):
        for k in range(DQ // 16):
            stage_v[r, pl.ds(k * 16, 16)] = zeros16
        return 0

    def _scale(rb, _):
        dv = dis_v[pl.ds(rb * 16, 16)]
        for i in range(16):
            sc = dv[i]
            for k in range(DQ // 16):
                stage_v[rb * 16 + i, pl.ds(k * 16, 16)] = (
                    stage_v[rb * 16 + i, pl.ds(k * 16, 16)] * sc)
        return 0

    def _zero_acc():
        lax.fori_loop(0, ROWS_PT, _zero_stage, 0)
        pltpu.sync_copy(stage_v, acc_sh.at[pl.ds(base, ROWS_PT)])

    _zero_acc()
    for p in range(2):
        q = 2 * c + p
        lax.fori_loop(0, ROWS_PT, _zero_stage, 0)

        @pl.when(s < 15)
        def _():
            pltpu.sync_copy(x_cat.at[pl.ds(q * N_NODES + base, ROWS_PT)],
                            stage_v)

        @pl.when(s == 15)
        def _():
            pltpu.sync_copy(x_cat.at[pl.ds(q * N_NODES + 15 * ROWS_PT,
                                           N_NODES - 15 * ROWS_PT)],
                            stage_v.at[pl.ds(0, N_NODES - 15 * ROWS_PT)])

        lax.fori_loop(0, ROWS_PT // 16, _scale, 0)
        pltpu.sync_copy(stage_v, xs.at[pl.ds(q * NPAD + base, ROWS_PT)])
    plsc.subcore_barrier()

    def _pass(table, q_off, out1, out2, last):
        _mk_src(q_off)

        def _blk(j, _):
            pltpu.async_copy(table.at[src_v.at[j]], gbuf, sem).wait()
            pltpu.sync_copy(gbuf, acc_sh.at[dst_v.at[j]], add=True)
            return 0
        lax.fori_loop(0, BPT, _blk, 0)
        plsc.subcore_barrier()

        pltpu.sync_copy(acc_sh.at[pl.ds(base, ROWS_PT)], stage_v)
        lax.fori_loop(0, ROWS_PT // 16, _scale, 0)
        pltpu.sync_copy(stage_v, out1.at[pl.ds(q_off + base, ROWS_PT)])
        if out2 is not None:
            lax.fori_loop(0, ROWS_PT // 16, _scale, 0)
            pltpu.sync_copy(stage_v, out2.at[pl.ds(q_off + base, ROWS_PT)])
        if not last:
            _zero_acc()
        plsc.subcore_barrier()

    for p in range(2):
        _pass(xs, (2 * c + p) * NPAD, h1, t1, last=False)
    for p in range(2):
        _pass(t1, (2 * c + p) * NPAD, h2, None, last=(p == 1))


def _make_sc_kernel():
    mesh = plsc.VectorSubcoreMesh(core_axis_name="c", subcore_axis_name="s",
                                  num_cores=2, num_subcores=16)
    hbm_out = jax.ShapeDtypeStruct((4 * NPAD, DQ), jnp.float32)
    deg_out = jax.ShapeDtypeStruct((32, NPAD), jnp.float32)
    return pl.kernel(
        _sc_body,
        out_type=(hbm_out, hbm_out, hbm_out, hbm_out, deg_out),
        mesh=mesh,
        compiler_params=pltpu.CompilerParams(needs_layout_passes=False,
                                             use_tc_tiling_on_sc=False),
        scratch_types=[
            pltpu.VMEM((BPT, B), jnp.int32),
            pltpu.VMEM((BPT, B), jnp.int32),
            pltpu.VMEM((BPT, B), jnp.int32),
            pltpu.VMEM((B, DQ), jnp.float32),
            pltpu.VMEM((ROWS_PT,), jnp.float32),
            pltpu.VMEM((NPAD,), jnp.float32),
            pltpu.VMEM((ROWS_PT, DQ), jnp.float32),
            pltpu.VMEM((ROWS_PT,), jnp.float32),
            pltpu.VMEM_SHARED((NPAD, DQ), jnp.float32),
            pltpu.SemaphoreType.DMA,
        ],
    )


def _combine(alpha, x, h1, h2):
    R = 2000
    half = lambda i: (i, 0)
    qspec = pl.BlockSpec((R, DQ), half)

    def body(alpha_ref, x_ref, h1a, h1b, h1c, h1d, h2a, h2b, h2c, h2d, o_ref):
        a0 = alpha_ref[0]
        a1 = alpha_ref[1]
        a2 = alpha_ref[2]
        h1q = (h1a, h1b, h1c, h1d)
        h2q = (h2a, h2b, h2c, h2d)
        for q in range(4):
            o_ref[:, q * DQ:(q + 1) * DQ] = (
                a0 * x_ref[:, q * DQ:(q + 1) * DQ]
                + a1 * h1q[q][...] + a2 * h2q[q][...])

    h1q = [h1[q * NPAD:q * NPAD + N_NODES] for q in range(4)]
    h2q = [h2[q * NPAD:q * NPAD + N_NODES] for q in range(4)]
    return pl.pallas_call(
        body,
        grid=(N_NODES // R,),
        in_specs=[pl.BlockSpec(memory_space=pltpu.SMEM),
                  pl.BlockSpec((R, D), half)] + [qspec] * 8,
        out_specs=pl.BlockSpec((R, D), half),
        out_shape=jax.ShapeDtypeStruct((N_NODES, D), jnp.float32),
    )(alpha, x, *h1q, *h2q)


def kernel(x, edge_index, alpha):
    ei = edge_index.astype(jnp.int32)
    pad = jnp.full((2, E_PAD - ei.shape[1]), PAD_NODE, jnp.int32)
    ei = jnp.concatenate([ei, pad], axis=1)
    edges = (ei[0] | (ei[1] << 14)).reshape(16 * BPT, B)
    x_cat = jnp.concatenate([x[:, q * DQ:(q + 1) * DQ] for q in range(4)],
                            axis=0)
    xs, t1, h1, h2, degp = _make_sc_kernel()(x_cat, edges)
    del xs, t1, degp
    return _combine(alpha, x, h1, h2)
